# baseline (speedup 1.0000x reference)
import jax
import jax.numpy as jnp
from jax.experimental import pallas as pl

_TOKEN_BLOCK = 256


def _router_body(x_ref, w_ref, b_ref, o_ref):
    o_ref[...] = jax.lax.dot_general(
        x_ref[...], w_ref[...],
        dimension_numbers=(((1,), (1,)), ((), ())),
        preferred_element_type=jnp.float32,
    ) + b_ref[...]


def kernel(x, W, b):
    tokens, d = x.shape
    n_experts = W.shape[0]
    blk = _TOKEN_BLOCK
    return pl.pallas_call(
        _router_body,
        grid=(tokens // blk,),
        in_specs=[
            pl.BlockSpec((blk, d), lambda i: (i, 0)),
            pl.BlockSpec((n_experts, d), lambda i: (0, 0)),
            pl.BlockSpec((1, n_experts), lambda i: (0, 0)),
        ],
        out_specs=pl.BlockSpec((blk, n_experts), lambda i: (i, 0)),
        out_shape=jax.ShapeDtypeStruct((tokens, n_experts), jnp.float32),
    )(x, W, b.reshape(1, n_experts))


# speedup vs baseline: 1.1778x; 1.1778x over previous
import jax
import jax.numpy as jnp
from jax.experimental import pallas as pl

_TOKEN_BLOCK = 512


def _router_body(x_ref, w_ref, b_ref, o_ref):
    o_ref[...] = jax.lax.dot_general(
        x_ref[...], w_ref[...],
        dimension_numbers=(((1,), (1,)), ((), ())),
        preferred_element_type=jnp.float32,
    ) + b_ref[...]


def kernel(x, W, b):
    tokens, d = x.shape
    n_experts = W.shape[0]
    blk = _TOKEN_BLOCK
    return pl.pallas_call(
        _router_body,
        grid=(tokens // blk,),
        in_specs=[
            pl.BlockSpec((blk, d), lambda i: (i, 0)),
            pl.BlockSpec((n_experts, d), lambda i: (0, 0)),
            pl.BlockSpec((1, n_experts), lambda i: (0, 0)),
        ],
        out_specs=pl.BlockSpec((blk, n_experts), lambda i: (i, 0)),
        out_shape=jax.ShapeDtypeStruct((tokens, n_experts), jnp.float32),
    )(x, W, b.reshape(1, n_experts))
